# baseline (speedup 1.0000x reference)
import functools

import jax
import jax.numpy as jnp
from jax import lax
from jax.experimental import pallas as pl
from jax.experimental.pallas import tpu as pltpu
from jax.experimental.pallas import tpu_sc as plsc

N = 10000
E = 160000
NC = 2
NS = 16
NW = NC * NS
NP = 10240
EP = 163840
EPB = EP // 128
BLKS_PER_W = EP // NW // 128
ROWS_PER_SUB = NP // NS
BN = 1280
NB = NP // BN
C = 128


def _make_sc_pass(has_ea, NBUF):
    CH = 64
    mesh = plsc.VectorSubcoreMesh(
        core_axis_name="c", subcore_axis_name="s", num_cores=NC,
        num_subcores=NS)
    NCH = (EP // NW) // CH
    NSR = (EP // NW) // 128
    scratch = [
        pltpu.VMEM((NSR, 128), jnp.int32),
        pltpu.VMEM((NCH, CH), jnp.int32),
    ]
    scratch += [pltpu.VMEM((CH, C), jnp.float32)] * NBUF
    if has_ea:
        scratch += [pltpu.VMEM((CH, C), jnp.float32)] * NBUF
    scratch += [
        pltpu.VMEM_SHARED((NP, C), jnp.float32),
    ]
    scratch += [pltpu.SemaphoreType.DMA] * (2 * NBUF)

    def body(feat, src2, dst2, *rest):
        if has_ea:
            ea = rest[0]
            rest = rest[1:]
        zeros, out = rest[0], rest[1]
        rest = rest[2:]
        srcb, dstb = rest[0], rest[1]
        rest = rest[2:]
        gb = rest[:NBUF]
        rest = rest[NBUF:]
        if has_ea:
            eb = rest[:NBUF]
            rest = rest[NBUF:]
        acc = rest[0]
        gsem = rest[1:1 + NBUF]
        ssem = rest[1 + NBUF:1 + 2 * NBUF]

        cid = lax.axis_index("c")
        sid = lax.axis_index("s")
        w = sid * NC + cid
        r_sub = sid * ROWS_PER_SUB
        base = w * NCH
        pltpu.sync_copy(zeros.at[pl.ds(r_sub, ROWS_PER_SUB)],
                        acc.at[pl.ds(r_sub, ROWS_PER_SUB)])
        pltpu.sync_copy(src2.at[pl.ds(w * NSR, NSR)], srcb)
        pltpu.sync_copy(dst2.at[pl.ds(base, NCH)], dstb)
        plsc.subcore_barrier()

        def start_in(c, b):
            gidx = srcb.at[c // 2, pl.ds((c % 2) * CH, CH)]
            pltpu.async_copy(feat.at[gidx], gb[b], gsem[b])
            if has_ea:
                pltpu.async_copy(ea.at[pl.ds((base + c) * CH, CH)],
                                 eb[b], gsem[b])

        def wait_in(b):
            pltpu.make_async_copy(feat.at[srcb.at[0, pl.ds(0, CH)]],
                                  gb[b], gsem[b]).wait()
            if has_ea:
                pltpu.make_async_copy(ea.at[pl.ds(0, CH)], eb[b],
                                      gsem[b]).wait()

        def relu(b):
            def rbody(r, c2):
                for cb in range(96 // 16):
                    sl = pl.ds(cb * 16, 16)
                    gb[b][r, sl] = jnp.maximum(
                        gb[b][r, sl] + eb[b][r, sl], 0.0)
                return c2

            lax.fori_loop(0, CH, rbody, 0)

        def start_scatter(c, b):
            pltpu.async_copy(gb[b], acc.at[dstb.at[c]], ssem[b], add=True)

        def wait_scatter(b):
            pltpu.make_async_copy(gb[b], acc.at[dstb.at[0]], ssem[b]).wait()

        start_in(0, 0)

        def slot(c, k, start_next):
            b = k % NBUF
            bn = (k + 1) % NBUF
            wait_in(b)
            if has_ea:
                relu(b)
            start_scatter(c, b)
            if start_next:

                @pl.when(c >= NBUF - 1)
                def _():
                    wait_scatter(bn)

                start_in(c + 1, bn)

        def pipe_body(g, carry):
            c0 = NBUF * g
            for k in range(NBUF):
                slot(c0 + k, k, True)
            return carry

        lax.fori_loop(0, NCH // NBUF - 1, pipe_body, 0)
        for k in range(NBUF):
            c = NCH - NBUF + k
            slot(c, c % NBUF, k < NBUF - 1)
        for b in range(NBUF):
            wait_scatter(b)
        plsc.subcore_barrier()
        pltpu.sync_copy(acc.at[pl.ds(r_sub, ROWS_PER_SUB)],
                        out.at[cid, pl.ds(r_sub, ROWS_PER_SUB)])

    fn = pl.kernel(
        body,
        out_type=jax.ShapeDtypeStruct((2, NP, C), jnp.float32),
        mesh=mesh,
        scratch_types=scratch,
    )
    return fn


_sc_pass = _make_sc_pass(has_ea=False, NBUF=4)
_sc_pass_ea = _make_sc_pass(has_ea=True, NBUF=2)


def _enc_l1_body(s_ref, w1_ref, b1_ref, w2_ref, b2_ref, op_ref, om_ref):
    sc = s_ref[...]
    w1 = w1_ref[...]
    b1 = b1_ref[...]
    op_ref[...] = jnp.dot(
        jnp.maximum(sc * w1 + b1, 0.0), w2_ref[...],
        preferred_element_type=jnp.float32) + b2_ref[...]
    om_ref[...] = jnp.dot(
        jnp.maximum(-sc * w1 + b1, 0.0), w2_ref[...],
        preferred_element_type=jnp.float32) + b2_ref[...]


def _enc_l1(s_col, w1p, b1p, w2p, b2p):
    spec = pl.BlockSpec((BN, C), lambda i: (i, 0))
    return pl.pallas_call(
        _enc_l1_body,
        grid=(NB,),
        in_specs=[
            pl.BlockSpec((BN, 1), lambda i: (i, 0)),
            pl.BlockSpec((1, C), lambda i: (0, 0)),
            pl.BlockSpec((1, C), lambda i: (0, 0)),
            pl.BlockSpec((C, C), lambda i: (0, 0)),
            pl.BlockSpec((1, C), lambda i: (0, 0)),
        ],
        out_specs=[spec, spec],
        out_shape=[jax.ShapeDtypeStruct((NP, C), jnp.float32)] * 2,
    )(s_col, w1p, b1p, w2p, b2p)


def _enc_l2_body(*refs):
    us = refs[0:8]
    accs = refs[8:16]
    (w1_ref, b1_ref, w2_ref, b2_ref, r1w_ref, r1b_ref, r2w_ref, r2b_ref,
     ew_ref, eb_ref, o_ref) = refs[16:]
    gs = []
    for k in range(4):
        g = None
        for sgn in range(2):
            u = us[2 * k + sgn][...]
            a = accs[2 * k + sgn][...]
            t = u + a[0] + a[1]
            hid = jnp.maximum(
                jnp.dot(t, w1_ref[...], preferred_element_type=jnp.float32)
                + b1_ref[...], 0.0)
            o = jnp.dot(hid, w2_ref[...],
                        preferred_element_type=jnp.float32) + b2_ref[...]
            g = o if g is None else g + o
        gs.append(g)
    g = jnp.concatenate(gs, axis=1)
    hr = jnp.maximum(
        jnp.dot(g, r1w_ref[...], preferred_element_type=jnp.float32)
        + r1b_ref[...], 0.0)
    p_enc = jnp.dot(hr, r2w_ref[...],
                    preferred_element_type=jnp.float32) + r2b_ref[...]
    o_ref[...] = jnp.dot(p_enc, ew_ref[...],
                         preferred_element_type=jnp.float32) + eb_ref[...]


def _enc_l2(us, accs, w1p, b1p, w2p, b2p, r1w, r1b, r2w, r2b, ew, eb):
    in_specs = (
        [pl.BlockSpec((BN, C), lambda i: (i, 0))] * 8
        + [pl.BlockSpec((2, BN, C), lambda i: (0, i, 0))] * 8
        + [
            pl.BlockSpec((C, C), lambda i: (0, 0)),
            pl.BlockSpec((1, C), lambda i: (0, 0)),
            pl.BlockSpec((C, 16), lambda i: (0, 0)),
            pl.BlockSpec((1, 16), lambda i: (0, 0)),
            pl.BlockSpec((64, C), lambda i: (0, 0)),
            pl.BlockSpec((1, C), lambda i: (0, 0)),
            pl.BlockSpec((C, 16), lambda i: (0, 0)),
            pl.BlockSpec((1, 16), lambda i: (0, 0)),
            pl.BlockSpec((16, C), lambda i: (0, 0)),
            pl.BlockSpec((1, C), lambda i: (0, 0)),
        ])
    return pl.pallas_call(
        _enc_l2_body,
        grid=(NB,),
        in_specs=in_specs,
        out_specs=pl.BlockSpec((BN, C), lambda i: (i, 0)),
        out_shape=jax.ShapeDtypeStruct((NP, C), jnp.float32),
    )(*us, *accs, w1p, b1p, w2p, b2p, r1w, r1b, r2w, r2b, ew, eb)


def _gine_body(residual, h_ref, a_ref, w1_ref, b1_ref, w2_ref, b2_ref, o_ref):
    h = h_ref[...]
    t = h + a_ref[0] + a_ref[1]
    hid = jnp.maximum(
        jnp.dot(t, w1_ref[...], preferred_element_type=jnp.float32)
        + b1_ref[...], 0.0)
    hn = jnp.dot(hid, w2_ref[...],
                 preferred_element_type=jnp.float32) + b2_ref[...]
    o_ref[...] = h + hn if residual else hn


def _gine_mlp(h, acc, w1p, b1p, w2p, b2p, residual):
    co = w2p.shape[1]
    return pl.pallas_call(
        functools.partial(_gine_body, residual),
        grid=(NB,),
        in_specs=[
            pl.BlockSpec((BN, C), lambda i: (i, 0)),
            pl.BlockSpec((2, BN, C), lambda i: (0, i, 0)),
            pl.BlockSpec((C, C), lambda i: (0, 0)),
            pl.BlockSpec((1, C), lambda i: (0, 0)),
            pl.BlockSpec((C, co), lambda i: (0, 0)),
            pl.BlockSpec((1, co), lambda i: (0, 0)),
        ],
        out_specs=pl.BlockSpec((BN, co), lambda i: (i, 0)),
        out_shape=jax.ShapeDtypeStruct((NP, co), jnp.float32),
    )(h, acc, w1p, b1p, w2p, b2p)


def _readout_body(h_ref, w0_ref, b0_ref, w1_ref, b1_ref, w2_ref, b2_ref,
                  o_ref, acc_ref):
    i = pl.program_id(0)

    @pl.when(i == 0)
    def _():
        acc_ref[...] = jnp.zeros_like(acc_ref)

    row = lax.broadcasted_iota(jnp.int32, (BN, 16), 0) + i * BN
    blk = jnp.where(row < N, h_ref[...], 0.0)
    acc_ref[...] += jnp.sum(blk, axis=0, keepdims=True)

    @pl.when(i == NB - 1)
    def _():
        y = acc_ref[...] / float(N)
        y = jnp.maximum(jnp.dot(y, w0_ref[...],
                                preferred_element_type=jnp.float32)
                        + b0_ref[...], 0.0)
        y = jnp.maximum(jnp.dot(y, w1_ref[...],
                                preferred_element_type=jnp.float32)
                        + b1_ref[...], 0.0)
        o_ref[...] = jnp.dot(y, w2_ref[...],
                             preferred_element_type=jnp.float32) + b2_ref[...]


def _readout(hl, ro):
    def padw(p, ci, co):
        w = jnp.zeros((16, 16), jnp.float32).at[:ci, :co].set(p["w"])
        b = jnp.zeros((1, 16), jnp.float32).at[0, :co].set(p["b"])
        return w, b

    w0, b0 = padw(ro[0], 10, 5)
    w1, b1 = padw(ro[1], 5, 2)
    w2, b2 = padw(ro[2], 2, 1)
    wspec = pl.BlockSpec((16, 16), lambda i: (0, 0))
    bspec = pl.BlockSpec((1, 16), lambda i: (0, 0))
    out = pl.pallas_call(
        _readout_body,
        grid=(NB,),
        in_specs=[pl.BlockSpec((BN, 16), lambda i: (i, 0)),
                  wspec, bspec, wspec, bspec, wspec, bspec],
        out_specs=pl.BlockSpec((1, 16), lambda i: (0, 0)),
        out_shape=jax.ShapeDtypeStruct((1, 16), jnp.float32),
        scratch_shapes=[pltpu.VMEM((1, 16), jnp.float32)],
    )(hl, w0, b0, w1, b1, w2, b2)
    return out[0, 0:1]


def _pad_mlp2(p, ci, h, co, co_pad):
    w1 = jnp.zeros((C, C), jnp.float32).at[:ci, :h].set(p["l1"]["w"])
    b1 = jnp.zeros((1, C), jnp.float32).at[0, :h].set(p["l1"]["b"])
    w2 = jnp.zeros((C, co_pad), jnp.float32).at[:h, :co].set(p["l2"]["w"])
    b2 = jnp.zeros((1, co_pad), jnp.float32).at[0, :co].set(p["l2"]["b"])
    return w1, b1, w2, b2


def kernel(x, edge_index, edge_attr, params):
    src = edge_index[0]
    dst = edge_index[1]
    pad_e = EP - E
    padi = (jnp.arange(pad_e, dtype=jnp.int32) % 8) + N
    src_f = jnp.concatenate([src, padi])
    dst_f = jnp.concatenate([dst, padi])
    src2 = src_f.reshape(EP // 128, 128)
    dst2 = dst_f.reshape(EP // 64, 64)
    ea_p = jnp.pad(edge_attr, ((0, pad_e), (0, C - edge_attr.shape[1])))
    x_p = jnp.pad(x, ((0, NP - N), (0, C - x.shape[1])))
    zc = jnp.zeros((NP, C), jnp.float32)

    accx = _sc_pass(x_p, src2, dst2, zc)
    s = x_p + accx[0] + accx[1]

    enc0, enc1 = params["enc"]
    e0w1 = jnp.zeros((1, C), jnp.float32).at[:, :95].set(enc0["l1"]["w"])
    e0b1 = jnp.zeros((1, C), jnp.float32).at[0, :95].set(enc0["l1"]["b"])
    e0w2 = jnp.zeros((C, C), jnp.float32).at[:95, :95].set(enc0["l2"]["w"])
    e0b2 = jnp.zeros((1, C), jnp.float32).at[0, :95].set(enc0["l2"]["b"])

    us = []
    for k in range(4):
        up, um = _enc_l1(s[:, k:k + 1], e0w1, e0b1, e0w2, e0b2)
        us += [up, um]
    accs = [_sc_pass(u, src2, dst2, zc) for u in us]

    e1w1, e1b1, e1w2, e1b2 = _pad_mlp2(enc1, 95, 95, 4, 16)
    r1w = jnp.zeros((4, 16, C), jnp.float32).at[:, :4, :95].set(
        params["rho"][0]["w"].reshape(4, 4, 95)).reshape(64, C)
    r1b = jnp.zeros((1, C), jnp.float32).at[0, :95].set(params["rho"][0]["b"])
    r2w = jnp.zeros((C, 16), jnp.float32).at[:95, :4].set(params["rho"][1]["w"])
    r2b = jnp.zeros((1, 16), jnp.float32).at[0, :4].set(params["rho"][1]["b"])
    ew = jnp.zeros((16, C), jnp.float32).at[:4, :95].set(params["emb"]["w"])
    eb = jnp.zeros((1, C), jnp.float32).at[0, :95].set(params["emb"]["b"])
    h = _enc_l2(us, accs, e1w1, e1b1, e1w2, e1b2,
                r1w, r1b, r2w, r2b, ew, eb)

    ngine = len(params["gine"])
    for li, p in enumerate(params["gine"]):
        last = li == ngine - 1
        acch = _sc_pass_ea(h, src2, dst2, ea_p, zc)
        if last:
            w1p, b1p, w2p, b2p = _pad_mlp2(p, 95, 95, 10, 16)
        else:
            w1p, b1p, w2p, b2p = _pad_mlp2(p, 95, 95, 95, C)
        h = _gine_mlp(h, acch, w1p, b1p, w2p, b2p, residual=not last)

    return _readout(h, params["ro"])


# speedup vs baseline: 1.0507x; 1.0507x over previous
import functools

import jax
import jax.numpy as jnp
from jax import lax
from jax.experimental import pallas as pl
from jax.experimental.pallas import tpu as pltpu
from jax.experimental.pallas import tpu_sc as plsc

N = 10000
E = 160000
NC = 2
NS = 16
NW = NC * NS
NP = 10240
EP = 163840
EPB = EP // 128
BLKS_PER_W = EP // NW // 128
ROWS_PER_SUB = NP // NS
BN = 1280
NB = NP // BN
C = 128


def _make_sc_pass(has_ea, NBUF, CH=64):
    mesh = plsc.VectorSubcoreMesh(
        core_axis_name="c", subcore_axis_name="s", num_cores=NC,
        num_subcores=NS)
    NCH = (EP // NW) // CH
    NSR = (EP // NW) // 128
    scratch = [
        pltpu.VMEM((NSR, 128), jnp.int32),
        pltpu.VMEM((NCH, CH), jnp.int32),
    ]
    scratch += [pltpu.VMEM((CH, C), jnp.float32)] * NBUF
    if has_ea:
        scratch += [pltpu.VMEM((CH, C), jnp.float32)] * NBUF
    scratch += [
        pltpu.VMEM_SHARED((NP, C), jnp.float32),
    ]
    scratch += [pltpu.SemaphoreType.DMA] * (2 * NBUF)

    def body(feat, src2, dst2, *rest):
        if has_ea:
            ea = rest[0]
            rest = rest[1:]
        zeros, out = rest[0], rest[1]
        rest = rest[2:]
        srcb, dstb = rest[0], rest[1]
        rest = rest[2:]
        gb = rest[:NBUF]
        rest = rest[NBUF:]
        if has_ea:
            eb = rest[:NBUF]
            rest = rest[NBUF:]
        acc = rest[0]
        gsem = rest[1:1 + NBUF]
        ssem = rest[1 + NBUF:1 + 2 * NBUF]

        cid = lax.axis_index("c")
        sid = lax.axis_index("s")
        w = sid * NC + cid
        r_sub = sid * ROWS_PER_SUB
        base = w * NCH
        pltpu.sync_copy(zeros.at[pl.ds(r_sub, ROWS_PER_SUB)],
                        acc.at[pl.ds(r_sub, ROWS_PER_SUB)])
        pltpu.sync_copy(src2.at[pl.ds(w * NSR, NSR)], srcb)
        pltpu.sync_copy(dst2.at[pl.ds(base, NCH)], dstb)
        plsc.subcore_barrier()

        def gidx(c):
            if CH == 128:
                return srcb.at[c]
            return srcb.at[c // 2, pl.ds((c % 2) * CH, CH)]

        def start_in(c, b):
            pltpu.async_copy(feat.at[gidx(c)], gb[b], gsem[b])
            if has_ea:
                pltpu.async_copy(ea.at[pl.ds((base + c) * CH, CH)],
                                 eb[b], gsem[b])

        def wait_in(b):
            pltpu.make_async_copy(feat.at[gidx(0)], gb[b], gsem[b]).wait()
            if has_ea:
                pltpu.make_async_copy(ea.at[pl.ds(0, CH)], eb[b],
                                      gsem[b]).wait()

        def relu(b):
            def rbody(r, c2):
                for cb in range(96 // 16):
                    sl = pl.ds(cb * 16, 16)
                    gb[b][r, sl] = jnp.maximum(
                        gb[b][r, sl] + eb[b][r, sl], 0.0)
                return c2

            lax.fori_loop(0, CH, rbody, 0)

        def start_scatter(c, b):
            pltpu.async_copy(gb[b], acc.at[dstb.at[c]], ssem[b], add=True)

        def wait_scatter(b):
            pltpu.make_async_copy(gb[b], acc.at[dstb.at[0]], ssem[b]).wait()

        start_in(0, 0)

        def slot(c, k, start_next):
            b = k % NBUF
            bn = (k + 1) % NBUF
            wait_in(b)
            if has_ea:
                relu(b)
            start_scatter(c, b)
            if start_next:

                @pl.when(c >= NBUF - 1)
                def _():
                    wait_scatter(bn)

                start_in(c + 1, bn)

        def pipe_body(g, carry):
            c0 = NBUF * g
            for k in range(NBUF):
                slot(c0 + k, k, True)
            return carry

        lax.fori_loop(0, NCH // NBUF - 1, pipe_body, 0)
        for k in range(NBUF):
            c = NCH - NBUF + k
            slot(c, c % NBUF, k < NBUF - 1)
        for b in range(NBUF):
            wait_scatter(b)
        plsc.subcore_barrier()
        pltpu.sync_copy(acc.at[pl.ds(r_sub, ROWS_PER_SUB)],
                        out.at[cid, pl.ds(r_sub, ROWS_PER_SUB)])

    fn = pl.kernel(
        body,
        out_type=jax.ShapeDtypeStruct((2, NP, C), jnp.float32),
        mesh=mesh,
        scratch_types=scratch,
    )
    return fn


_sc_pass = _make_sc_pass(has_ea=False, NBUF=2, CH=128)
_sc_pass_ea = _make_sc_pass(has_ea=True, NBUF=2, CH=64)


def _enc_l1_body(s_ref, w1_ref, b1_ref, w2_ref, b2_ref, op_ref, om_ref):
    sc = s_ref[...]
    w1 = w1_ref[...]
    b1 = b1_ref[...]
    op_ref[...] = jnp.dot(
        jnp.maximum(sc * w1 + b1, 0.0), w2_ref[...],
        preferred_element_type=jnp.float32) + b2_ref[...]
    om_ref[...] = jnp.dot(
        jnp.maximum(-sc * w1 + b1, 0.0), w2_ref[...],
        preferred_element_type=jnp.float32) + b2_ref[...]


def _enc_l1(s_col, w1p, b1p, w2p, b2p):
    spec = pl.BlockSpec((BN, C), lambda i: (i, 0))
    return pl.pallas_call(
        _enc_l1_body,
        grid=(NB,),
        in_specs=[
            pl.BlockSpec((BN, 1), lambda i: (i, 0)),
            pl.BlockSpec((1, C), lambda i: (0, 0)),
            pl.BlockSpec((1, C), lambda i: (0, 0)),
            pl.BlockSpec((C, C), lambda i: (0, 0)),
            pl.BlockSpec((1, C), lambda i: (0, 0)),
        ],
        out_specs=[spec, spec],
        out_shape=[jax.ShapeDtypeStruct((NP, C), jnp.float32)] * 2,
    )(s_col, w1p, b1p, w2p, b2p)


def _enc_l2_body(*refs):
    us = refs[0:8]
    accs = refs[8:16]
    (w1_ref, b1_ref, w2_ref, b2_ref, r1w_ref, r1b_ref, r2w_ref, r2b_ref,
     ew_ref, eb_ref, o_ref) = refs[16:]
    gs = []
    for k in range(4):
        g = None
        for sgn in range(2):
            u = us[2 * k + sgn][...]
            a = accs[2 * k + sgn][...]
            t = u + a[0] + a[1]
            hid = jnp.maximum(
                jnp.dot(t, w1_ref[...], preferred_element_type=jnp.float32)
                + b1_ref[...], 0.0)
            o = jnp.dot(hid, w2_ref[...],
                        preferred_element_type=jnp.float32) + b2_ref[...]
            g = o if g is None else g + o
        gs.append(g)
    g = jnp.concatenate(gs, axis=1)
    hr = jnp.maximum(
        jnp.dot(g, r1w_ref[...], preferred_element_type=jnp.float32)
        + r1b_ref[...], 0.0)
    p_enc = jnp.dot(hr, r2w_ref[...],
                    preferred_element_type=jnp.float32) + r2b_ref[...]
    o_ref[...] = jnp.dot(p_enc, ew_ref[...],
                         preferred_element_type=jnp.float32) + eb_ref[...]


def _enc_l2(us, accs, w1p, b1p, w2p, b2p, r1w, r1b, r2w, r2b, ew, eb):
    in_specs = (
        [pl.BlockSpec((BN, C), lambda i: (i, 0))] * 8
        + [pl.BlockSpec((2, BN, C), lambda i: (0, i, 0))] * 8
        + [
            pl.BlockSpec((C, C), lambda i: (0, 0)),
            pl.BlockSpec((1, C), lambda i: (0, 0)),
            pl.BlockSpec((C, 16), lambda i: (0, 0)),
            pl.BlockSpec((1, 16), lambda i: (0, 0)),
            pl.BlockSpec((64, C), lambda i: (0, 0)),
            pl.BlockSpec((1, C), lambda i: (0, 0)),
            pl.BlockSpec((C, 16), lambda i: (0, 0)),
            pl.BlockSpec((1, 16), lambda i: (0, 0)),
            pl.BlockSpec((16, C), lambda i: (0, 0)),
            pl.BlockSpec((1, C), lambda i: (0, 0)),
        ])
    return pl.pallas_call(
        _enc_l2_body,
        grid=(NB,),
        in_specs=in_specs,
        out_specs=pl.BlockSpec((BN, C), lambda i: (i, 0)),
        out_shape=jax.ShapeDtypeStruct((NP, C), jnp.float32),
    )(*us, *accs, w1p, b1p, w2p, b2p, r1w, r1b, r2w, r2b, ew, eb)


def _gine_body(residual, h_ref, a_ref, w1_ref, b1_ref, w2_ref, b2_ref, o_ref):
    h = h_ref[...]
    t = h + a_ref[0] + a_ref[1]
    hid = jnp.maximum(
        jnp.dot(t, w1_ref[...], preferred_element_type=jnp.float32)
        + b1_ref[...], 0.0)
    hn = jnp.dot(hid, w2_ref[...],
                 preferred_element_type=jnp.float32) + b2_ref[...]
    o_ref[...] = h + hn if residual else hn


def _gine_mlp(h, acc, w1p, b1p, w2p, b2p, residual):
    co = w2p.shape[1]
    return pl.pallas_call(
        functools.partial(_gine_body, residual),
        grid=(NB,),
        in_specs=[
            pl.BlockSpec((BN, C), lambda i: (i, 0)),
            pl.BlockSpec((2, BN, C), lambda i: (0, i, 0)),
            pl.BlockSpec((C, C), lambda i: (0, 0)),
            pl.BlockSpec((1, C), lambda i: (0, 0)),
            pl.BlockSpec((C, co), lambda i: (0, 0)),
            pl.BlockSpec((1, co), lambda i: (0, 0)),
        ],
        out_specs=pl.BlockSpec((BN, co), lambda i: (i, 0)),
        out_shape=jax.ShapeDtypeStruct((NP, co), jnp.float32),
    )(h, acc, w1p, b1p, w2p, b2p)


def _readout_body(h_ref, w0_ref, b0_ref, w1_ref, b1_ref, w2_ref, b2_ref,
                  o_ref, acc_ref):
    i = pl.program_id(0)

    @pl.when(i == 0)
    def _():
        acc_ref[...] = jnp.zeros_like(acc_ref)

    row = lax.broadcasted_iota(jnp.int32, (BN, 16), 0) + i * BN
    blk = jnp.where(row < N, h_ref[...], 0.0)
    acc_ref[...] += jnp.sum(blk, axis=0, keepdims=True)

    @pl.when(i == NB - 1)
    def _():
        y = acc_ref[...] / float(N)
        y = jnp.maximum(jnp.dot(y, w0_ref[...],
                                preferred_element_type=jnp.float32)
                        + b0_ref[...], 0.0)
        y = jnp.maximum(jnp.dot(y, w1_ref[...],
                                preferred_element_type=jnp.float32)
                        + b1_ref[...], 0.0)
        o_ref[...] = jnp.dot(y, w2_ref[...],
                             preferred_element_type=jnp.float32) + b2_ref[...]


def _readout(hl, ro):
    def padw(p, ci, co):
        w = jnp.zeros((16, 16), jnp.float32).at[:ci, :co].set(p["w"])
        b = jnp.zeros((1, 16), jnp.float32).at[0, :co].set(p["b"])
        return w, b

    w0, b0 = padw(ro[0], 10, 5)
    w1, b1 = padw(ro[1], 5, 2)
    w2, b2 = padw(ro[2], 2, 1)
    wspec = pl.BlockSpec((16, 16), lambda i: (0, 0))
    bspec = pl.BlockSpec((1, 16), lambda i: (0, 0))
    out = pl.pallas_call(
        _readout_body,
        grid=(NB,),
        in_specs=[pl.BlockSpec((BN, 16), lambda i: (i, 0)),
                  wspec, bspec, wspec, bspec, wspec, bspec],
        out_specs=pl.BlockSpec((1, 16), lambda i: (0, 0)),
        out_shape=jax.ShapeDtypeStruct((1, 16), jnp.float32),
        scratch_shapes=[pltpu.VMEM((1, 16), jnp.float32)],
    )(hl, w0, b0, w1, b1, w2, b2)
    return out[0, 0:1]


def _pad_mlp2(p, ci, h, co, co_pad):
    w1 = jnp.zeros((C, C), jnp.float32).at[:ci, :h].set(p["l1"]["w"])
    b1 = jnp.zeros((1, C), jnp.float32).at[0, :h].set(p["l1"]["b"])
    w2 = jnp.zeros((C, co_pad), jnp.float32).at[:h, :co].set(p["l2"]["w"])
    b2 = jnp.zeros((1, co_pad), jnp.float32).at[0, :co].set(p["l2"]["b"])
    return w1, b1, w2, b2


def kernel(x, edge_index, edge_attr, params):
    src = edge_index[0]
    dst = edge_index[1]
    pad_e = EP - E
    padi = (jnp.arange(pad_e, dtype=jnp.int32) % 8) + N
    src_f = jnp.concatenate([src, padi])
    dst_f = jnp.concatenate([dst, padi])
    src2 = src_f.reshape(EP // 128, 128)
    dst2 = dst_f.reshape(EP // 128, 128)
    dst2e = dst_f.reshape(EP // 64, 64)
    ea_p = jnp.pad(edge_attr, ((0, pad_e), (0, C - edge_attr.shape[1])))
    x_p = jnp.pad(x, ((0, NP - N), (0, C - x.shape[1])))
    zc = jnp.zeros((NP, C), jnp.float32)

    accx = _sc_pass(x_p, src2, dst2, zc)
    s = x_p + accx[0] + accx[1]

    enc0, enc1 = params["enc"]
    e0w1 = jnp.zeros((1, C), jnp.float32).at[:, :95].set(enc0["l1"]["w"])
    e0b1 = jnp.zeros((1, C), jnp.float32).at[0, :95].set(enc0["l1"]["b"])
    e0w2 = jnp.zeros((C, C), jnp.float32).at[:95, :95].set(enc0["l2"]["w"])
    e0b2 = jnp.zeros((1, C), jnp.float32).at[0, :95].set(enc0["l2"]["b"])

    us = []
    for k in range(4):
        up, um = _enc_l1(s[:, k:k + 1], e0w1, e0b1, e0w2, e0b2)
        us += [up, um]
    accs = [_sc_pass(u, src2, dst2, zc) for u in us]

    e1w1, e1b1, e1w2, e1b2 = _pad_mlp2(enc1, 95, 95, 4, 16)
    r1w = jnp.zeros((4, 16, C), jnp.float32).at[:, :4, :95].set(
        params["rho"][0]["w"].reshape(4, 4, 95)).reshape(64, C)
    r1b = jnp.zeros((1, C), jnp.float32).at[0, :95].set(params["rho"][0]["b"])
    r2w = jnp.zeros((C, 16), jnp.float32).at[:95, :4].set(params["rho"][1]["w"])
    r2b = jnp.zeros((1, 16), jnp.float32).at[0, :4].set(params["rho"][1]["b"])
    ew = jnp.zeros((16, C), jnp.float32).at[:4, :95].set(params["emb"]["w"])
    eb = jnp.zeros((1, C), jnp.float32).at[0, :95].set(params["emb"]["b"])
    h = _enc_l2(us, accs, e1w1, e1b1, e1w2, e1b2,
                r1w, r1b, r2w, r2b, ew, eb)

    ngine = len(params["gine"])
    for li, p in enumerate(params["gine"]):
        last = li == ngine - 1
        acch = _sc_pass_ea(h, src2, dst2e, ea_p, zc)
        if last:
            w1p, b1p, w2p, b2p = _pad_mlp2(p, 95, 95, 10, 16)
        else:
            w1p, b1p, w2p, b2p = _pad_mlp2(p, 95, 95, 95, C)
        h = _gine_mlp(h, acch, w1p, b1p, w2p, b2p, residual=not last)

    return _readout(h, params["ro"])


# speedup vs baseline: 1.0986x; 1.0456x over previous
import functools

import jax
import jax.numpy as jnp
from jax import lax
from jax.experimental import pallas as pl
from jax.experimental.pallas import tpu as pltpu
from jax.experimental.pallas import tpu_sc as plsc

N = 10000
E = 160000
NC = 2
NS = 16
NW = NC * NS
NP = 10240
EP = 163840
EPW = EP // NW
ROWS_PER_SUB = NP // NS
BN = 1280
NB = NP // BN
C = 128


def _make_sc_pass(has_ea, CH, NG):
    NBUF = 2
    mesh = plsc.VectorSubcoreMesh(
        core_axis_name="c", subcore_axis_name="s", num_cores=NC,
        num_subcores=NS)
    NCH = EPW // CH
    scratch = [
        pltpu.VMEM((NCH, CH), jnp.int32),
        pltpu.VMEM((NCH, CH), jnp.int32),
    ]
    if NG > 1:
        scratch.append(pltpu.VMEM((NCH, CH), jnp.int32))
    scratch += [pltpu.VMEM((CH, C), jnp.float32)] * NBUF
    if has_ea:
        scratch.append(pltpu.VMEM((CH, C), jnp.float32))
    scratch += [
        pltpu.VMEM_SHARED((NP, C), jnp.float32),
    ]
    scratch += [pltpu.SemaphoreType.DMA] * (2 * NBUF + (1 if has_ea else 0))

    def body(feat, src2, dst2, *rest):
        if has_ea:
            ea = rest[0]
            rest = rest[1:]
        zeros, out = rest[0], rest[1]
        rest = rest[2:]
        srcb, dstb = rest[0], rest[1]
        rest = rest[2:]
        if NG > 1:
            srcg = rest[0]
            rest = rest[1:]
        else:
            srcg = srcb
        gb = rest[:NBUF]
        rest = rest[NBUF:]
        if has_ea:
            eb = rest[0]
            rest = rest[1:]
        acc = rest[0]
        gsem = rest[1:1 + NBUF]
        ssem = rest[1 + NBUF:1 + 2 * NBUF]
        esem = rest[1 + 2 * NBUF] if has_ea else None

        cid = lax.axis_index("c")
        sid = lax.axis_index("s")
        w = sid * NC + cid
        r_sub = sid * ROWS_PER_SUB
        base = w * NCH
        pltpu.sync_copy(src2.at[pl.ds(base, NCH)], srcb)
        pltpu.sync_copy(dst2.at[pl.ds(base, NCH)], dstb)

        def start_in(c, b):
            pltpu.async_copy(feat.at[srcg.at[c]], gb[b], gsem[b])

        def wait_in(b):
            pltpu.make_async_copy(feat.at[srcg.at[0]], gb[b], gsem[b]).wait()

        def start_ea(c):
            pltpu.async_copy(ea.at[pl.ds((base + c) * CH, CH)], eb, esem)

        def wait_ea():
            pltpu.make_async_copy(ea.at[pl.ds(0, CH)], eb, esem).wait()

        def relu(b):
            def rbody(r, c2):
                for cb in range(96 // 16):
                    sl = pl.ds(cb * 16, 16)
                    gb[b][r, sl] = jnp.maximum(
                        gb[b][r, sl] + eb[r, sl], 0.0)
                return c2

            lax.fori_loop(0, CH, rbody, 0)

        def start_scatter(c, b):
            pltpu.async_copy(gb[b], acc.at[dstb.at[c]], ssem[b], add=True)

        def wait_scatter(b):
            pltpu.make_async_copy(gb[b], acc.at[dstb.at[0]], ssem[b]).wait()

        def one_group(g):
            pltpu.sync_copy(zeros.at[pl.ds(r_sub, ROWS_PER_SUB)],
                            acc.at[pl.ds(r_sub, ROWS_PER_SUB)])
            if NG > 1:
                off = g * NP

                def shift(r, c2):
                    for cb in range(CH // 16):
                        sl = pl.ds(cb * 16, 16)
                        srcg[r, sl] = srcb[r, sl] + off
                    return c2

                lax.fori_loop(0, NCH, shift, 0)
            plsc.subcore_barrier()

            start_in(0, 0)
            if has_ea:
                start_ea(0)

            def slot(c, k, start_next):
                b = k % NBUF
                bn = (k + 1) % NBUF
                wait_in(b)
                if has_ea:
                    wait_ea()
                    relu(b)
                    if start_next:
                        start_ea(c + 1)
                start_scatter(c, b)
                if start_next:

                    @pl.when(c >= NBUF - 1)
                    def _():
                        wait_scatter(bn)

                    start_in(c + 1, bn)

            def pipe_body(gg, carry):
                c0 = NBUF * gg
                for k in range(NBUF):
                    slot(c0 + k, k, True)
                return carry

            lax.fori_loop(0, NCH // NBUF - 1, pipe_body, 0)
            for k in range(NBUF):
                c = NCH - NBUF + k
                slot(c, c % NBUF, k < NBUF - 1)
            for b in range(NBUF):
                wait_scatter(b)
            plsc.subcore_barrier()
            orow = (g * 2 + cid) * NP + r_sub
            pltpu.sync_copy(acc.at[pl.ds(r_sub, ROWS_PER_SUB)],
                            out.at[pl.ds(orow, ROWS_PER_SUB)])

        if NG == 1:
            one_group(0)
        else:
            lax.fori_loop(0, NG, lambda g, c2: (one_group(g), c2)[1], 0)

    fn = pl.kernel(
        body,
        out_type=jax.ShapeDtypeStruct((NG * 2 * NP, C), jnp.float32),
        mesh=mesh,
        scratch_types=scratch,
    )
    return fn


_sc_pass = _make_sc_pass(has_ea=False, CH=128, NG=1)
_sc_enc = _make_sc_pass(has_ea=False, CH=128, NG=8)
_sc_pass_ea = _make_sc_pass(has_ea=True, CH=64, NG=1)


def _enc_l1_body(s_ref, w1_ref, b1_ref, w2_ref, b2_ref, o_ref):
    p = pl.program_id(0)
    sgn = jnp.where(p % 2 == 0, 1.0, -1.0)
    sc = s_ref[0] * sgn
    h = jnp.maximum(sc * w1_ref[...] + b1_ref[...], 0.0)
    o_ref[...] = jnp.dot(h, w2_ref[...],
                         preferred_element_type=jnp.float32) + b2_ref[...]


def _enc_l1(s3, w1p, b1p, w2p, b2p):
    return pl.pallas_call(
        _enc_l1_body,
        grid=(8, NB),
        in_specs=[
            pl.BlockSpec((1, BN, 1), lambda p, i: (p // 2, i, 0)),
            pl.BlockSpec((1, C), lambda p, i: (0, 0)),
            pl.BlockSpec((1, C), lambda p, i: (0, 0)),
            pl.BlockSpec((C, C), lambda p, i: (0, 0)),
            pl.BlockSpec((1, C), lambda p, i: (0, 0)),
        ],
        out_specs=pl.BlockSpec((BN, C), lambda p, i: (p * NB + i, 0)),
        out_shape=jax.ShapeDtypeStruct((8 * NP, C), jnp.float32),
    )(s3, w1p, b1p, w2p, b2p)


def _enc_l2_body(us_ref, acc_ref, w1_ref, b1_ref, w2_ref, b2_ref,
                 r1w_ref, r1b_ref, r2w_ref, r2b_ref, ew_ref, eb_ref, o_ref):
    gs = []
    for k in range(4):
        g = None
        for sgn in range(2):
            p = 2 * k + sgn
            t = us_ref[p] + acc_ref[p, 0] + acc_ref[p, 1]
            hid = jnp.maximum(
                jnp.dot(t, w1_ref[...], preferred_element_type=jnp.float32)
                + b1_ref[...], 0.0)
            o = jnp.dot(hid, w2_ref[...],
                        preferred_element_type=jnp.float32) + b2_ref[...]
            g = o if g is None else g + o
        gs.append(g)
    g = jnp.concatenate(gs, axis=1)
    hr = jnp.maximum(
        jnp.dot(g, r1w_ref[...], preferred_element_type=jnp.float32)
        + r1b_ref[...], 0.0)
    p_enc = jnp.dot(hr, r2w_ref[...],
                    preferred_element_type=jnp.float32) + r2b_ref[...]
    o_ref[...] = jnp.dot(p_enc, ew_ref[...],
                         preferred_element_type=jnp.float32) + eb_ref[...]


def _enc_l2(u8, acc8, w1p, b1p, w2p, b2p, r1w, r1b, r2w, r2b, ew, eb):
    in_specs = [
        pl.BlockSpec((8, BN, C), lambda i: (0, i, 0)),
        pl.BlockSpec((8, 2, BN, C), lambda i: (0, 0, i, 0)),
        pl.BlockSpec((C, C), lambda i: (0, 0)),
        pl.BlockSpec((1, C), lambda i: (0, 0)),
        pl.BlockSpec((C, 16), lambda i: (0, 0)),
        pl.BlockSpec((1, 16), lambda i: (0, 0)),
        pl.BlockSpec((64, C), lambda i: (0, 0)),
        pl.BlockSpec((1, C), lambda i: (0, 0)),
        pl.BlockSpec((C, 16), lambda i: (0, 0)),
        pl.BlockSpec((1, 16), lambda i: (0, 0)),
        pl.BlockSpec((16, C), lambda i: (0, 0)),
        pl.BlockSpec((1, C), lambda i: (0, 0)),
    ]
    return pl.pallas_call(
        _enc_l2_body,
        grid=(NB,),
        in_specs=in_specs,
        out_specs=pl.BlockSpec((BN, C), lambda i: (i, 0)),
        out_shape=jax.ShapeDtypeStruct((NP, C), jnp.float32),
    )(u8, acc8, w1p, b1p, w2p, b2p, r1w, r1b, r2w, r2b, ew, eb)


def _gine_body(residual, h_ref, a_ref, w1_ref, b1_ref, w2_ref, b2_ref, o_ref):
    h = h_ref[...]
    t = h + a_ref[0] + a_ref[1]
    hid = jnp.maximum(
        jnp.dot(t, w1_ref[...], preferred_element_type=jnp.float32)
        + b1_ref[...], 0.0)
    hn = jnp.dot(hid, w2_ref[...],
                 preferred_element_type=jnp.float32) + b2_ref[...]
    o_ref[...] = h + hn if residual else hn


def _gine_mlp(h, acc, w1p, b1p, w2p, b2p, residual):
    co = w2p.shape[1]
    return pl.pallas_call(
        functools.partial(_gine_body, residual),
        grid=(NB,),
        in_specs=[
            pl.BlockSpec((BN, C), lambda i: (i, 0)),
            pl.BlockSpec((2, BN, C), lambda i: (0, i, 0)),
            pl.BlockSpec((C, C), lambda i: (0, 0)),
            pl.BlockSpec((1, C), lambda i: (0, 0)),
            pl.BlockSpec((C, co), lambda i: (0, 0)),
            pl.BlockSpec((1, co), lambda i: (0, 0)),
        ],
        out_specs=pl.BlockSpec((BN, co), lambda i: (i, 0)),
        out_shape=jax.ShapeDtypeStruct((NP, co), jnp.float32),
    )(h, acc, w1p, b1p, w2p, b2p)


def _readout_body(h_ref, w0_ref, b0_ref, w1_ref, b1_ref, w2_ref, b2_ref,
                  o_ref, acc_ref):
    i = pl.program_id(0)

    @pl.when(i == 0)
    def _():
        acc_ref[...] = jnp.zeros_like(acc_ref)

    row = lax.broadcasted_iota(jnp.int32, (BN, 16), 0) + i * BN
    blk = jnp.where(row < N, h_ref[...], 0.0)
    acc_ref[...] += jnp.sum(blk, axis=0, keepdims=True)

    @pl.when(i == NB - 1)
    def _():
        y = acc_ref[...] / float(N)
        y = jnp.maximum(jnp.dot(y, w0_ref[...],
                                preferred_element_type=jnp.float32)
                        + b0_ref[...], 0.0)
        y = jnp.maximum(jnp.dot(y, w1_ref[...],
                                preferred_element_type=jnp.float32)
                        + b1_ref[...], 0.0)
        o_ref[...] = jnp.dot(y, w2_ref[...],
                             preferred_element_type=jnp.float32) + b2_ref[...]


def _readout(hl, ro):
    def padw(p, ci, co):
        w = jnp.zeros((16, 16), jnp.float32).at[:ci, :co].set(p["w"])
        b = jnp.zeros((1, 16), jnp.float32).at[0, :co].set(p["b"])
        return w, b

    w0, b0 = padw(ro[0], 10, 5)
    w1, b1 = padw(ro[1], 5, 2)
    w2, b2 = padw(ro[2], 2, 1)
    wspec = pl.BlockSpec((16, 16), lambda i: (0, 0))
    bspec = pl.BlockSpec((1, 16), lambda i: (0, 0))
    out = pl.pallas_call(
        _readout_body,
        grid=(NB,),
        in_specs=[pl.BlockSpec((BN, 16), lambda i: (i, 0)),
                  wspec, bspec, wspec, bspec, wspec, bspec],
        out_specs=pl.BlockSpec((1, 16), lambda i: (0, 0)),
        out_shape=jax.ShapeDtypeStruct((1, 16), jnp.float32),
        scratch_shapes=[pltpu.VMEM((1, 16), jnp.float32)],
    )(hl, w0, b0, w1, b1, w2, b2)
    return out[0, 0:1]


def _pad_mlp2(p, ci, h, co, co_pad):
    w1 = jnp.zeros((C, C), jnp.float32).at[:ci, :h].set(p["l1"]["w"])
    b1 = jnp.zeros((1, C), jnp.float32).at[0, :h].set(p["l1"]["b"])
    w2 = jnp.zeros((C, co_pad), jnp.float32).at[:h, :co].set(p["l2"]["w"])
    b2 = jnp.zeros((1, co_pad), jnp.float32).at[0, :co].set(p["l2"]["b"])
    return w1, b1, w2, b2


def kernel(x, edge_index, edge_attr, params):
    src = edge_index[0]
    dst = edge_index[1]
    pad_e = EP - E
    padi = (jnp.arange(pad_e, dtype=jnp.int32) % 8) + N
    src_f = jnp.concatenate([src, padi])
    dst_f = jnp.concatenate([dst, padi])
    src2 = src_f.reshape(EP // 128, 128)
    dst2 = dst_f.reshape(EP // 128, 128)
    src2e = src_f.reshape(EP // 64, 64)
    dst2e = dst_f.reshape(EP // 64, 64)
    ea_p = jnp.pad(edge_attr, ((0, pad_e), (0, C - edge_attr.shape[1])))
    x_p = jnp.pad(x, ((0, NP - N), (0, C - x.shape[1])))
    zc = jnp.zeros((NP, C), jnp.float32)

    accx = _sc_pass(x_p, src2, dst2, zc).reshape(2, NP, C)
    s = x_p + accx[0] + accx[1]

    enc0, enc1 = params["enc"]
    e0w1 = jnp.zeros((1, C), jnp.float32).at[:, :95].set(enc0["l1"]["w"])
    e0b1 = jnp.zeros((1, C), jnp.float32).at[0, :95].set(enc0["l1"]["b"])
    e0w2 = jnp.zeros((C, C), jnp.float32).at[:95, :95].set(enc0["l2"]["w"])
    e0b2 = jnp.zeros((1, C), jnp.float32).at[0, :95].set(enc0["l2"]["b"])

    s3 = s[:, :4].T.reshape(4, NP, 1)
    u8 = _enc_l1(s3, e0w1, e0b1, e0w2, e0b2)
    acc8 = _sc_enc(u8, src2, dst2, zc)

    e1w1, e1b1, e1w2, e1b2 = _pad_mlp2(enc1, 95, 95, 4, 16)
    r1w = jnp.zeros((4, 16, C), jnp.float32).at[:, :4, :95].set(
        params["rho"][0]["w"].reshape(4, 4, 95)).reshape(64, C)
    r1b = jnp.zeros((1, C), jnp.float32).at[0, :95].set(params["rho"][0]["b"])
    r2w = jnp.zeros((C, 16), jnp.float32).at[:95, :4].set(params["rho"][1]["w"])
    r2b = jnp.zeros((1, 16), jnp.float32).at[0, :4].set(params["rho"][1]["b"])
    ew = jnp.zeros((16, C), jnp.float32).at[:4, :95].set(params["emb"]["w"])
    eb = jnp.zeros((1, C), jnp.float32).at[0, :95].set(params["emb"]["b"])
    h = _enc_l2(u8.reshape(8, NP, C), acc8.reshape(8, 2, NP, C),
                e1w1, e1b1, e1w2, e1b2,
                r1w, r1b, r2w, r2b, ew, eb)

    ngine = len(params["gine"])
    for li, p in enumerate(params["gine"]):
        last = li == ngine - 1
        acch = _sc_pass_ea(h, src2e, dst2e, ea_p, zc).reshape(2, NP, C)
        if last:
            w1p, b1p, w2p, b2p = _pad_mlp2(p, 95, 95, 10, 16)
        else:
            w1p, b1p, w2p, b2p = _pad_mlp2(p, 95, 95, 95, C)
        h = _gine_mlp(h, acch, w1p, b1p, w2p, b2p, residual=not last)

    return _readout(h, params["ro"])
